# baseline (speedup 1.0000x reference)
import functools

import jax
import jax.numpy as jnp
from jax import lax
from jax.experimental import pallas as pl
from jax.experimental.pallas import tpu as pltpu
from jax.experimental.pallas import tpu_sc as plsc

NC = 2
NS = 16
NW = NC * NS

B = 4096
C = 200
D = 32
BPW = B // NW
NG = 13
CP = NG * 16
G1 = 104
G2 = C - G1


def _fpmc_body(urows_h, lrows_h, cand_h, nemb_h, out_h,
               cidx_v, ubuf, lbuf, gbuf0, gbuf1, outbuf,
               sem_g0, sem_g1, sem_u):
    wid = lax.axis_index("s") * NC + lax.axis_index("c")
    dbase = pl.multiple_of(wid * (BPW * D), 8)
    cbase = pl.multiple_of(wid * (BPW * C), 8)

    pltpu.sync_copy(cand_h.at[pl.ds(cbase, BPW * C)], cidx_v)
    cu = pltpu.async_copy(urows_h.at[pl.ds(dbase, BPW * D)], ubuf, sem_u)
    cl = pltpu.async_copy(lrows_h.at[pl.ds(dbase, BPW * D)], lbuf, sem_u)
    cu.wait()
    cl.wait()

    lane = lax.iota(jnp.int32, 16)
    lane32 = lane * D
    zero16i = jnp.zeros((16,), jnp.int32)
    tail_mask = lane < (C - (NG - 1) * 16)
    col_idx = [jnp.full((16,), d, jnp.int32) for d in range(16)]

    def fire(b, gbuf, sem):
        offc = pl.multiple_of(b * C, 8)
        pltpu.async_copy(nemb_h.at[cidx_v.at[pl.ds(offc, G1)]],
                         gbuf.at[pl.ds(0, G1)], sem)
        pltpu.async_copy(
            nemb_h.at[cidx_v.at[pl.ds(pl.multiple_of(offc + G1, 8), G2)]],
            gbuf.at[pl.ds(G1, G2)], sem)

    def wait_fire(gbuf, sem):
        pltpu.make_async_copy(nemb_h.at[cidx_v.at[pl.ds(0, G1)]],
                              gbuf.at[pl.ds(0, G1)], sem).wait()
        pltpu.make_async_copy(nemb_h.at[cidx_v.at[pl.ds(0, G2)]],
                              gbuf.at[pl.ds(G1, G2)], sem).wait()

    def compute(b, gbuf):
        bd = pl.multiple_of(b * D, 8)
        bb = jnp.full((16,), bd, jnp.int32) + lane
        u0 = plsc.load_gather(ubuf, [bb])
        u1 = plsc.load_gather(ubuf, [bb + 16])
        l0 = plsc.load_gather(lbuf, [bb])
        l1 = plsc.load_gather(lbuf, [bb + 16])
        s = jnp.sum(u0 * l0 + u1 * l1)
        w0 = u0 + l0
        w1 = u1 + l1
        sv = jnp.full((16,), s, jnp.float32)

        accs = [sv] * NG
        for d in range(D):
            wv = w0 if d < 16 else w1
            wd = wv.at[col_idx[d % 16]].get(mode="promise_in_bounds")
            for grp in range(NG):
                flat = lane32 + (grp * 16 * D + d)
                g = plsc.load_gather(gbuf, [zero16i, flat])
                accs[grp] = accs[grp] + wd * g

        ob = pl.multiple_of(b * C, 8)
        for grp in range(NG):
            if grp < NG - 1:
                outbuf[pl.ds(pl.multiple_of(ob + grp * 16, 8), 16)] = \
                    accs[grp]
            else:
                plsc.store_compressed(
                    outbuf.at[pl.ds(pl.multiple_of(ob + grp * 16, 8), 16)],
                    accs[grp], mask=tail_mask)

    fire(0, gbuf0, sem_g0)

    def body(g, carry):
        b0 = g * 2
        b1 = b0 + 1
        fire(b1, gbuf1, sem_g1)
        wait_fire(gbuf0, sem_g0)
        compute(b0, gbuf0)
        fire(jnp.minimum(b0 + 2, BPW - 1), gbuf0, sem_g0)
        wait_fire(gbuf1, sem_g1)
        compute(b1, gbuf1)
        return carry

    lax.fori_loop(0, BPW // 2, body, 0)
    wait_fire(gbuf0, sem_g0)

    pltpu.sync_copy(outbuf.at[pl.ds(0, BPW * C)],
                    out_h.at[pl.ds(cbase, BPW * C)])


@functools.partial(
    pl.kernel,
    out_type=jax.ShapeDtypeStruct((B * C,), jnp.float32),
    mesh=plsc.VectorSubcoreMesh(core_axis_name="c", subcore_axis_name="s"),
    compiler_params=pltpu.CompilerParams(
        needs_layout_passes=False, use_tc_tiling_on_sc=False),
    scratch_types=[
        pltpu.VMEM((BPW * C,), jnp.int32),
        pltpu.VMEM((BPW * D,), jnp.float32),
        pltpu.VMEM((BPW * D,), jnp.float32),
        pltpu.VMEM((CP, D), jnp.float32),
        pltpu.VMEM((CP, D), jnp.float32),
        pltpu.VMEM((BPW * C + 8,), jnp.float32),
        pltpu.SemaphoreType.DMA,
        pltpu.SemaphoreType.DMA,
        pltpu.SemaphoreType.DMA,
    ],
)
def _fpmc_sc(urows_h, lrows_h, cand_h, nemb_h, out_h,
             cidx_v, ubuf, lbuf, gbuf0, gbuf1, outbuf,
             sem_g0, sem_g1, sem_u):
    _fpmc_body(urows_h, lrows_h, cand_h, nemb_h, out_h,
               cidx_v, ubuf, lbuf, gbuf0, gbuf1, outbuf,
               sem_g0, sem_g1, sem_u)


@jax.jit
def kernel(user, last, candidates, user_emb, last_item_emb, next_item_emb):
    user = user.reshape(B).astype(jnp.int32)
    last = last.reshape(B).astype(jnp.int32)
    candidates = candidates.reshape(B * C).astype(jnp.int32)
    u_rows = jnp.take(user_emb, user, axis=0).reshape(B * D)
    l_rows = jnp.take(last_item_emb, last, axis=0).reshape(B * D)
    out = _fpmc_sc(u_rows, l_rows, candidates, next_item_emb)
    return out.reshape(B, C)


# speedup vs baseline: 1.0261x; 1.0261x over previous
import functools

import jax
import jax.numpy as jnp
from jax import lax
from jax.experimental import pallas as pl
from jax.experimental.pallas import tpu as pltpu
from jax.experimental.pallas import tpu_sc as plsc

NC = 2
NS = 16
NW = NC * NS

B = 4096
C = 200
D = 32
BPW = B // NW
NG = 13
CP = NG * 16
G1 = 104
G2 = C - G1


def _fpmc_body(urows_h, lrows_h, cand_h, nemb_h, out_h,
               cidx_v, ubuf, lbuf, gbuf0, gbuf1, outbuf,
               sem_g0, sem_g1, sem_u):
    wid = lax.axis_index("s") * NC + lax.axis_index("c")
    dbase = pl.multiple_of(wid * (BPW * D), 8)
    cbase = pl.multiple_of(wid * (BPW * C), 8)

    pltpu.sync_copy(cand_h.at[pl.ds(cbase, BPW * C)], cidx_v)
    cu = pltpu.async_copy(urows_h.at[pl.ds(dbase, BPW * D)], ubuf, sem_u)
    cl = pltpu.async_copy(lrows_h.at[pl.ds(dbase, BPW * D)], lbuf, sem_u)
    cu.wait()
    cl.wait()

    lane = lax.iota(jnp.int32, 16)
    lane32 = lane * D
    zero16i = jnp.zeros((16,), jnp.int32)
    tail_mask = lane < (C - (NG - 1) * 16)
    col_idx = [jnp.full((16,), d, jnp.int32) for d in range(16)]

    def fire(b, gbuf, sem):
        offc = pl.multiple_of(b * C, 8)
        pltpu.async_copy(nemb_h.at[cidx_v.at[pl.ds(offc, G1)]],
                         gbuf.at[pl.ds(0, G1)], sem)
        pltpu.async_copy(
            nemb_h.at[cidx_v.at[pl.ds(pl.multiple_of(offc + G1, 8), G2)]],
            gbuf.at[pl.ds(G1, G2)], sem)

    def wait_fire(gbuf, sem):
        pltpu.make_async_copy(nemb_h.at[cidx_v.at[pl.ds(0, G1)]],
                              gbuf.at[pl.ds(0, G1)], sem).wait()
        pltpu.make_async_copy(nemb_h.at[cidx_v.at[pl.ds(0, G2)]],
                              gbuf.at[pl.ds(G1, G2)], sem).wait()

    def compute(b, gbuf):
        bd = pl.multiple_of(b * D, 8)
        bb = jnp.full((16,), bd, jnp.int32) + lane
        u0 = plsc.load_gather(ubuf, [bb])
        u1 = plsc.load_gather(ubuf, [bb + 16])
        l0 = plsc.load_gather(lbuf, [bb])
        l1 = plsc.load_gather(lbuf, [bb + 16])
        s = jnp.sum(u0 * l0 + u1 * l1)
        w0 = u0 + l0
        w1 = u1 + l1
        sv = jnp.full((16,), s, jnp.float32)

        ob = pl.multiple_of(b * C, 8)
        for g0, g1 in ((0, 7), (7, NG)):
            accs = [sv] * (g1 - g0)
            for d in range(D):
                wv = w0 if d < 16 else w1
                wd = wv.at[col_idx[d % 16]].get(mode="promise_in_bounds")
                for i, grp in enumerate(range(g0, g1)):
                    flat = lane32 + (grp * 16 * D + d)
                    g = plsc.load_gather(gbuf, [zero16i, flat])
                    accs[i] = accs[i] + wd * g
            for i, grp in enumerate(range(g0, g1)):
                if grp < NG - 1:
                    outbuf[pl.ds(pl.multiple_of(ob + grp * 16, 8), 16)] = \
                        accs[i]
                else:
                    plsc.store_compressed(
                        outbuf.at[pl.ds(pl.multiple_of(ob + grp * 16, 8),
                                        16)],
                        accs[i], mask=tail_mask)

    fire(0, gbuf0, sem_g0)

    def body(g, carry):
        b0 = g * 2
        b1 = b0 + 1
        fire(b1, gbuf1, sem_g1)
        wait_fire(gbuf0, sem_g0)
        compute(b0, gbuf0)
        fire(jnp.minimum(b0 + 2, BPW - 1), gbuf0, sem_g0)
        wait_fire(gbuf1, sem_g1)
        compute(b1, gbuf1)
        return carry

    lax.fori_loop(0, BPW // 2, body, 0)
    wait_fire(gbuf0, sem_g0)

    pltpu.sync_copy(outbuf.at[pl.ds(0, BPW * C)],
                    out_h.at[pl.ds(cbase, BPW * C)])


@functools.partial(
    pl.kernel,
    out_type=jax.ShapeDtypeStruct((B * C,), jnp.float32),
    mesh=plsc.VectorSubcoreMesh(core_axis_name="c", subcore_axis_name="s"),
    compiler_params=pltpu.CompilerParams(
        needs_layout_passes=False, use_tc_tiling_on_sc=False),
    scratch_types=[
        pltpu.VMEM((BPW * C,), jnp.int32),
        pltpu.VMEM((BPW * D,), jnp.float32),
        pltpu.VMEM((BPW * D,), jnp.float32),
        pltpu.VMEM((CP, D), jnp.float32),
        pltpu.VMEM((CP, D), jnp.float32),
        pltpu.VMEM((BPW * C + 8,), jnp.float32),
        pltpu.SemaphoreType.DMA,
        pltpu.SemaphoreType.DMA,
        pltpu.SemaphoreType.DMA,
    ],
)
def _fpmc_sc(urows_h, lrows_h, cand_h, nemb_h, out_h,
             cidx_v, ubuf, lbuf, gbuf0, gbuf1, outbuf,
             sem_g0, sem_g1, sem_u):
    _fpmc_body(urows_h, lrows_h, cand_h, nemb_h, out_h,
               cidx_v, ubuf, lbuf, gbuf0, gbuf1, outbuf,
               sem_g0, sem_g1, sem_u)


@jax.jit
def kernel(user, last, candidates, user_emb, last_item_emb, next_item_emb):
    user = user.reshape(B).astype(jnp.int32)
    last = last.reshape(B).astype(jnp.int32)
    candidates = candidates.reshape(B * C).astype(jnp.int32)
    u_rows = jnp.take(user_emb, user, axis=0).reshape(B * D)
    l_rows = jnp.take(last_item_emb, last, axis=0).reshape(B * D)
    out = _fpmc_sc(u_rows, l_rows, candidates, next_item_emb)
    return out.reshape(B, C)


# speedup vs baseline: 1.3137x; 1.2803x over previous
import functools

import jax
import jax.numpy as jnp
from jax import lax
from jax.experimental import pallas as pl
from jax.experimental.pallas import tpu as pltpu
from jax.experimental.pallas import tpu_sc as plsc

NC = 2
NS = 16
NW = NC * NS

B = 4096
C = 200
D = 32
BPW = B // NW
NG = 13
CP = NG * 16
G1 = 104
G2 = C - G1
GBLKS = ((0, 7), (7, NG))


def _fpmc_body(urows_h, lrows_h, cand_h, nemb_h, out_h,
               cidx_v, ubuf, lbuf, gbuf0, gbuf1, wtab, outbuf,
               sem_g0, sem_g1, sem_u):
    wid = lax.axis_index("s") * NC + lax.axis_index("c")
    dbase = pl.multiple_of(wid * (BPW * D), 8)
    cbase = pl.multiple_of(wid * (BPW * C), 8)

    pltpu.sync_copy(cand_h.at[pl.ds(cbase, BPW * C)], cidx_v)
    cu = pltpu.async_copy(urows_h.at[pl.ds(dbase, BPW * D)], ubuf, sem_u)
    cl = pltpu.async_copy(lrows_h.at[pl.ds(dbase, BPW * D)], lbuf, sem_u)
    cu.wait()
    cl.wait()

    lane = lax.iota(jnp.int32, 16)
    tail_mask = lane < (C - (NG - 1) * 16)
    colrot = [(lane + d) & 31 for d in range(D)]

    def fire(b, gbuf, sem):
        offc = pl.multiple_of(b * C, 8)
        pltpu.async_copy(nemb_h.at[cidx_v.at[pl.ds(offc, G1)]],
                         gbuf.at[pl.ds(0, G1)], sem)
        pltpu.async_copy(
            nemb_h.at[cidx_v.at[pl.ds(pl.multiple_of(offc + G1, 8), G2)]],
            gbuf.at[pl.ds(G1, G2)], sem)

    def wait_fire(gbuf, sem):
        pltpu.make_async_copy(nemb_h.at[cidx_v.at[pl.ds(0, G1)]],
                              gbuf.at[pl.ds(0, G1)], sem).wait()
        pltpu.make_async_copy(nemb_h.at[cidx_v.at[pl.ds(0, G2)]],
                              gbuf.at[pl.ds(G1, G2)], sem).wait()

    def compute(b, gbuf):
        bd = pl.multiple_of(b * D, 8)
        bb = jnp.full((16,), bd, jnp.int32) + lane
        u0 = plsc.load_gather(ubuf, [bb])
        u1 = plsc.load_gather(ubuf, [bb + 16])
        l0 = plsc.load_gather(lbuf, [bb])
        l1 = plsc.load_gather(lbuf, [bb + 16])
        s = jnp.sum(u0 * l0 + u1 * l1)
        wtab[pl.ds(0, 16)] = u0 + l0
        wtab[pl.ds(16, 16)] = u1 + l1
        sv = jnp.full((16,), s, jnp.float32)

        ob = pl.multiple_of(b * C, 8)
        for g0, g1 in GBLKS:
            accs = [sv] * (g1 - g0)
            for d in range(D):
                wrot = plsc.load_gather(wtab, [colrot[d]])
                for i, grp in enumerate(range(g0, g1)):
                    row_idx = lane + (grp * 16)
                    g = plsc.load_gather(gbuf, [row_idx, colrot[d]])
                    accs[i] = accs[i] + wrot * g
            for i, grp in enumerate(range(g0, g1)):
                if grp < NG - 1:
                    outbuf[pl.ds(pl.multiple_of(ob + grp * 16, 8), 16)] = \
                        accs[i]
                else:
                    plsc.store_compressed(
                        outbuf.at[pl.ds(pl.multiple_of(ob + grp * 16, 8),
                                        16)],
                        accs[i], mask=tail_mask)

    fire(0, gbuf0, sem_g0)

    def body(g, carry):
        b0 = g * 2
        b1 = b0 + 1
        fire(b1, gbuf1, sem_g1)
        wait_fire(gbuf0, sem_g0)
        compute(b0, gbuf0)
        fire(jnp.minimum(b0 + 2, BPW - 1), gbuf0, sem_g0)
        wait_fire(gbuf1, sem_g1)
        compute(b1, gbuf1)
        return carry

    lax.fori_loop(0, BPW // 2, body, 0)
    wait_fire(gbuf0, sem_g0)

    pltpu.sync_copy(outbuf.at[pl.ds(0, BPW * C)],
                    out_h.at[pl.ds(cbase, BPW * C)])


@functools.partial(
    pl.kernel,
    out_type=jax.ShapeDtypeStruct((B * C,), jnp.float32),
    mesh=plsc.VectorSubcoreMesh(core_axis_name="c", subcore_axis_name="s"),
    compiler_params=pltpu.CompilerParams(
        needs_layout_passes=False, use_tc_tiling_on_sc=False),
    scratch_types=[
        pltpu.VMEM((BPW * C,), jnp.int32),
        pltpu.VMEM((BPW * D,), jnp.float32),
        pltpu.VMEM((BPW * D,), jnp.float32),
        pltpu.VMEM((CP, D), jnp.float32),
        pltpu.VMEM((CP, D), jnp.float32),
        pltpu.VMEM((D,), jnp.float32),
        pltpu.VMEM((BPW * C + 8,), jnp.float32),
        pltpu.SemaphoreType.DMA,
        pltpu.SemaphoreType.DMA,
        pltpu.SemaphoreType.DMA,
    ],
)
def _fpmc_sc(urows_h, lrows_h, cand_h, nemb_h, out_h,
             cidx_v, ubuf, lbuf, gbuf0, gbuf1, wtab, outbuf,
             sem_g0, sem_g1, sem_u):
    _fpmc_body(urows_h, lrows_h, cand_h, nemb_h, out_h,
               cidx_v, ubuf, lbuf, gbuf0, gbuf1, wtab, outbuf,
               sem_g0, sem_g1, sem_u)


@jax.jit
def kernel(user, last, candidates, user_emb, last_item_emb, next_item_emb):
    user = user.reshape(B).astype(jnp.int32)
    last = last.reshape(B).astype(jnp.int32)
    candidates = candidates.reshape(B * C).astype(jnp.int32)
    u_rows = jnp.take(user_emb, user, axis=0).reshape(B * D)
    l_rows = jnp.take(last_item_emb, last, axis=0).reshape(B * D)
    out = _fpmc_sc(u_rows, l_rows, candidates, next_item_emb)
    return out.reshape(B, C)
